# baseline (speedup 1.0000x reference)
import functools

import jax
import jax.numpy as jnp
from jax import lax
from jax.experimental import pallas as pl
from jax.experimental.pallas import tpu as pltpu
from jax.experimental.pallas import tpu_sc as plsc

BATCH = 16384
EMB = 64
PLOT_DIM = 384


def _make_sc_gather(B, D):
    info = plsc.get_sparse_core_info()
    NC, NS = info.num_cores, info.num_subcores
    NW = NC * NS
    b_per_w = B // NW
    mesh = plsc.VectorSubcoreMesh(core_axis_name="c", subcore_axis_name="s")

    @functools.partial(
        pl.kernel,
        mesh=mesh,
        compiler_params=pltpu.CompilerParams(use_tc_tiling_on_sc=False),
        out_type=(
            jax.ShapeDtypeStruct((B, D), jnp.float32),
            jax.ShapeDtypeStruct((B, D), jnp.float32),
        ),
        scratch_types=[
            pltpu.VMEM((b_per_w,), jnp.int32),
            pltpu.VMEM((b_per_w, D), jnp.float32),
            pltpu.VMEM((b_per_w,), jnp.int32),
            pltpu.VMEM((b_per_w, D), jnp.float32),
            pltpu.SemaphoreType.DMA,
            pltpu.SemaphoreType.DMA,
        ],
    )
    def gather_kernel(utab, uidx, mtab, midx, uout, mout,
                      uidx_v, urows_v, midx_v, mrows_v, usem, msem):
        wid = lax.axis_index("s") * NC + lax.axis_index("c")
        base = wid * b_per_w
        pltpu.sync_copy(uidx.at[pl.ds(base, b_per_w)], uidx_v)
        pltpu.sync_copy(midx.at[pl.ds(base, b_per_w)], midx_v)
        ucp = pltpu.async_copy(utab.at[uidx_v], urows_v, usem)
        mcp = pltpu.async_copy(mtab.at[midx_v], mrows_v, msem)
        ucp.wait()
        mcp.wait()
        pltpu.sync_copy(urows_v, uout.at[pl.ds(base, b_per_w)])
        pltpu.sync_copy(mrows_v, mout.at[pl.ds(base, b_per_w)])

    return gather_kernel


def _mlp_body(u_ref, m_ref, p_ref, w1u_ref, w1m_ref, w1p_ref, b1_ref,
              w2_ref, b2_ref, w3_ref, b3_ref, w4_ref, b4_ref, o_ref):
    x = (jnp.dot(u_ref[...], w1u_ref[...], preferred_element_type=jnp.float32)
         + jnp.dot(m_ref[...], w1m_ref[...], preferred_element_type=jnp.float32)
         + jnp.dot(p_ref[...], w1p_ref[...], preferred_element_type=jnp.float32)
         + b1_ref[...])
    x = jnp.maximum(x, 0.0)
    x = jnp.maximum(
        jnp.dot(x, w2_ref[...], preferred_element_type=jnp.float32) + b2_ref[...], 0.0)
    x = jnp.maximum(
        jnp.dot(x, w3_ref[...], preferred_element_type=jnp.float32) + b3_ref[...], 0.0)
    o_ref[...] = jnp.dot(x, w4_ref[...], preferred_element_type=jnp.float32) + b4_ref[...]


def _mlp(uemb, memb, plot, W1u, W1m, W1p, b1, W2, b2, W3, b3, W4, b4, block_rows):
    B = uemb.shape[0]
    grid = (B // block_rows,)

    def rows(i):
        return (i, 0)

    def whole(i):
        return (0, 0)

    return pl.pallas_call(
        _mlp_body,
        grid=grid,
        in_specs=[
            pl.BlockSpec((block_rows, EMB), rows),
            pl.BlockSpec((block_rows, EMB), rows),
            pl.BlockSpec((block_rows, PLOT_DIM), rows),
            pl.BlockSpec(W1u.shape, whole),
            pl.BlockSpec(W1m.shape, whole),
            pl.BlockSpec(W1p.shape, whole),
            pl.BlockSpec(b1.shape, whole),
            pl.BlockSpec(W2.shape, whole),
            pl.BlockSpec(b2.shape, whole),
            pl.BlockSpec(W3.shape, whole),
            pl.BlockSpec(b3.shape, whole),
            pl.BlockSpec(W4.shape, whole),
            pl.BlockSpec(b4.shape, whole),
        ],
        out_specs=pl.BlockSpec((block_rows, 1), rows),
        out_shape=jax.ShapeDtypeStruct((B, 1), jnp.float32),
    )(uemb, memb, plot, W1u, W1m, W1p, b1, W2, b2, W3, b3, W4, b4)


def kernel(users, movies, plot_embeddings, user_table, movie_table,
           W1, b1, W2, b2, W3, b3, W4, b4):
    u32 = users.astype(jnp.int32)
    m32 = movies.astype(jnp.int32)
    uemb, memb = _make_sc_gather(BATCH, EMB)(user_table, u32, movie_table, m32)
    W1u = W1[:EMB]
    W1m = W1[EMB:2 * EMB]
    W1p = W1[2 * EMB:]
    return _mlp(uemb, memb, plot_embeddings,
                W1u, W1m, W1p, b1.reshape(1, -1),
                W2, b2.reshape(1, -1), W3, b3.reshape(1, -1),
                W4, b4.reshape(1, -1), block_rows=2048)


# speedup vs baseline: 1.4937x; 1.4937x over previous
import functools

import jax
import jax.numpy as jnp
from jax import lax
from jax.experimental import pallas as pl
from jax.experimental.pallas import tpu as pltpu
from jax.experimental.pallas import tpu_sc as plsc

BATCH = 16384
EMB = 64
PLOT_DIM = 384


def _make_sc_gather(B, D):
    info = plsc.get_sparse_core_info()
    NC, NS = info.num_cores, info.num_subcores
    NW = NC * NS
    b_per_w = B // NW
    mesh = plsc.VectorSubcoreMesh(core_axis_name="c", subcore_axis_name="s")
    K = 8

    @functools.partial(
        pl.kernel,
        mesh=mesh,
        out_type=(
            jax.ShapeDtypeStruct((B, D), jnp.float32),
            jax.ShapeDtypeStruct((B, D), jnp.float32),
        ),
        scratch_types=[
            pltpu.VMEM((b_per_w,), jnp.int32),
            pltpu.VMEM((b_per_w, D), jnp.float32),
            pltpu.SemaphoreType.DMA,
        ],
    )
    def gather_kernel(utab, uidx, mtab, midx, uout, mout,
                      idx_v, rows_v, sem):
        wid = lax.axis_index("s") * NC + lax.axis_index("c")
        base = wid * b_per_w

        def gather_rows(tab, idx, out):
            pltpu.sync_copy(idx.at[pl.ds(base, b_per_w)], idx_v)

            def body(g, carry):
                vec = idx_v[pl.ds(g * 16, 16)]
                cps = []
                for j in range(16):
                    r = vec[j]
                    cps.append(pltpu.async_copy(
                        tab.at[r], rows_v.at[g * 16 + j], sem))
                for cp in cps:
                    cp.wait()
                return carry
            lax.fori_loop(0, b_per_w // 16, body, 0)
            pltpu.sync_copy(rows_v, out.at[pl.ds(base, b_per_w)])

        gather_rows(utab, uidx, uout)
        gather_rows(mtab, midx, mout)

    return gather_kernel


def _mlp_body(u_ref, m_ref, p_ref, w1u_ref, w1m_ref, w1p_ref, b1_ref,
              w2_ref, b2_ref, w3_ref, b3_ref, w4_ref, b4_ref, o_ref):
    x = (jnp.dot(u_ref[...], w1u_ref[...], preferred_element_type=jnp.float32)
         + jnp.dot(m_ref[...], w1m_ref[...], preferred_element_type=jnp.float32)
         + jnp.dot(p_ref[...], w1p_ref[...], preferred_element_type=jnp.float32)
         + b1_ref[...])
    x = jnp.maximum(x, 0.0)
    x = jnp.maximum(
        jnp.dot(x, w2_ref[...], preferred_element_type=jnp.float32) + b2_ref[...], 0.0)
    x = jnp.maximum(
        jnp.dot(x, w3_ref[...], preferred_element_type=jnp.float32) + b3_ref[...], 0.0)
    o_ref[...] = jnp.dot(x, w4_ref[...], preferred_element_type=jnp.float32) + b4_ref[...]


def _mlp(uemb, memb, plot, W1u, W1m, W1p, b1, W2, b2, W3, b3, W4, b4, block_rows):
    B = uemb.shape[0]
    grid = (B // block_rows,)

    def rows(i):
        return (i, 0)

    def whole(i):
        return (0, 0)

    return pl.pallas_call(
        _mlp_body,
        grid=grid,
        in_specs=[
            pl.BlockSpec((block_rows, EMB), rows),
            pl.BlockSpec((block_rows, EMB), rows),
            pl.BlockSpec((block_rows, PLOT_DIM), rows),
            pl.BlockSpec(W1u.shape, whole),
            pl.BlockSpec(W1m.shape, whole),
            pl.BlockSpec(W1p.shape, whole),
            pl.BlockSpec(b1.shape, whole),
            pl.BlockSpec(W2.shape, whole),
            pl.BlockSpec(b2.shape, whole),
            pl.BlockSpec(W3.shape, whole),
            pl.BlockSpec(b3.shape, whole),
            pl.BlockSpec(W4.shape, whole),
            pl.BlockSpec(b4.shape, whole),
        ],
        out_specs=pl.BlockSpec((block_rows, 1), rows),
        out_shape=jax.ShapeDtypeStruct((B, 1), jnp.float32),
    )(uemb, memb, plot, W1u, W1m, W1p, b1, W2, b2, W3, b3, W4, b4)


def kernel(users, movies, plot_embeddings, user_table, movie_table,
           W1, b1, W2, b2, W3, b3, W4, b4):
    u32 = users.astype(jnp.int32)
    m32 = movies.astype(jnp.int32)
    uemb, memb = _make_sc_gather(BATCH, EMB)(user_table, u32, movie_table, m32)
    W1u = W1[:EMB]
    W1m = W1[EMB:2 * EMB]
    W1p = W1[2 * EMB:]
    return _mlp(uemb, memb, plot_embeddings,
                W1u, W1m, W1p, b1.reshape(1, -1),
                W2, b2.reshape(1, -1), W3, b3.reshape(1, -1),
                W4, b4.reshape(1, -1), block_rows=2048)
